# Initial kernel scaffold; baseline (speedup 1.0000x reference)
import functools

import jax
import jax.numpy as jnp
from jax import lax
from jax.experimental import pallas as pl
from jax.experimental.pallas import tpu as pltpu
from jax.experimental.pallas import tpu_sc as plsc

N = 10000
E = 320000
D = 128
DE = 16

NC = 2
NS = 16
NW = NC * NS
K = 128
CH_PER_TILE = -(-E // (NW * K))
E_PAD = NW * CH_PER_TILE * K
N_PAD = 10016
ROWS_PER_TILE = N_PAD // NS
DUMMY = N


def _sc_scatter(x, src_p, dst_p, ea_p, zeros_big, zeros_se):
    mesh = plsc.VectorSubcoreMesh(core_axis_name="c", subcore_axis_name="s")

    @functools.partial(
        pl.kernel,
        out_type=(
            jax.ShapeDtypeStruct((NC, N_PAD, D), jnp.float32),
            jax.ShapeDtypeStruct((NC, N_PAD, DE), jnp.float32),
        ),
        mesh=mesh,
        scratch_types=[
            pltpu.VMEM((K,), jnp.int32),
            pltpu.VMEM((K,), jnp.int32),
            pltpu.VMEM((K, D), jnp.float32),
            pltpu.VMEM((K, DE), jnp.float32),
            pltpu.VMEM_SHARED((N_PAD, D), jnp.float32),
            pltpu.VMEM_SHARED((N_PAD, DE), jnp.float32),
            pltpu.SemaphoreType.DMA,
        ],
    )
    def sc_body(x_hbm, src_hbm, dst_hbm, ea_hbm, z_hbm, zse_hbm,
                agg_out, se_out, src_v, dst_v, rows_v, ea_v, agg_sh, se_sh, sem):
        cid = lax.axis_index("c")
        sid = lax.axis_index("s")
        wid = cid * NS + sid

        r0 = sid * ROWS_PER_TILE
        pltpu.sync_copy(z_hbm.at[pl.ds(r0, ROWS_PER_TILE)],
                        agg_sh.at[pl.ds(r0, ROWS_PER_TILE)])
        pltpu.sync_copy(zse_hbm.at[pl.ds(r0, ROWS_PER_TILE)],
                        se_sh.at[pl.ds(r0, ROWS_PER_TILE)])
        plsc.subcore_barrier()

        def chunk(i, carry):
            off = (wid * CH_PER_TILE + i) * K
            pltpu.sync_copy(src_hbm.at[pl.ds(off, K)], src_v)
            pltpu.sync_copy(dst_hbm.at[pl.ds(off, K)], dst_v)
            pltpu.sync_copy(ea_hbm.at[pl.ds(off, K)], ea_v)
            pltpu.async_copy(x_hbm.at[src_v], rows_v, sem).wait()
            pltpu.sync_copy(rows_v, agg_sh.at[dst_v], add=True)
            pltpu.sync_copy(ea_v, se_sh.at[dst_v], add=True)
            return carry

        lax.fori_loop(0, CH_PER_TILE, chunk, 0)
        plsc.subcore_barrier()

        pltpu.sync_copy(agg_sh.at[pl.ds(r0, ROWS_PER_TILE)],
                        agg_out.at[cid, pl.ds(r0, ROWS_PER_TILE)])
        pltpu.sync_copy(se_sh.at[pl.ds(r0, ROWS_PER_TILE)],
                        se_out.at[cid, pl.ds(r0, ROWS_PER_TILE)])

    return sc_body(x, src_p, dst_p, ea_p, zeros_big, zeros_se)


def _tc_body(aggp_ref, sep_ref, x_ref, We_ref, Wr_ref, br_ref, Wo_ref,
             g_ref, be_ref, out_ref):
    agg = aggp_ref[0, :N, :] + aggp_ref[1, :N, :]
    se = sep_ref[0, :N, :] + sep_ref[1, :N, :]
    x = x_ref[...]
    ea = lax.dot_general(se, We_ref[...], (((1,), (1,)), ((), ())),
                         preferred_element_type=jnp.float32)
    m = agg + ea
    pre = lax.dot_general(m, Wr_ref[...], (((1,), (1,)), ((), ())),
                          preferred_element_type=jnp.float32)
    pre = pre + lax.dot_general(x, Wo_ref[...], (((1,), (1,)), ((), ())),
                                preferred_element_type=jnp.float32)
    pre = pre + br_ref[...]
    pre = jnp.maximum(pre, 0.0)
    mean = jnp.mean(pre, axis=0, keepdims=True)
    var = jnp.mean((pre - mean) ** 2, axis=0, keepdims=True)
    out_ref[...] = (pre - mean) * lax.rsqrt(var + 1e-5) * g_ref[...] + be_ref[...]


def kernel(x, edge_index, edge_attr, W_edge, W_rel, b_rel, W_root, gamma, beta):
    src = edge_index[0].astype(jnp.int32)
    dst = edge_index[1].astype(jnp.int32)
    pad = E_PAD - E
    src_p = jnp.concatenate([src, jnp.zeros((pad,), jnp.int32)])
    dst_p = jnp.concatenate([dst, jnp.full((pad,), DUMMY, jnp.int32)])
    ea_p = jnp.concatenate([edge_attr, jnp.zeros((pad, DE), jnp.float32)])
    zeros_big = jnp.zeros((N_PAD, D), jnp.float32)
    zeros_se = jnp.zeros((N_PAD, DE), jnp.float32)

    aggp, sep = _sc_scatter(x, src_p, dst_p, ea_p, zeros_big, zeros_se)

    out = pl.pallas_call(
        _tc_body,
        out_shape=jax.ShapeDtypeStruct((N, D), jnp.float32),
    )(aggp, sep, x, W_edge, W_rel, b_rel.reshape(1, D), W_root,
      gamma.reshape(1, D), beta.reshape(1, D))
    return out


# speedup vs baseline: 2.5857x; 2.5857x over previous
import functools

import jax
import jax.numpy as jnp
from jax import lax
from jax.experimental import pallas as pl
from jax.experimental.pallas import tpu as pltpu
from jax.experimental.pallas import tpu_sc as plsc

N = 10000
E = 320000
D = 128
DE = 16

NC = 2
NS = 16
NW = NC * NS
K = 128
CH_PER_TILE = -(-E // (NW * K))
E_PAD = NW * CH_PER_TILE * K
N_PAD = 10112
ROWS_PER_TILE = N_PAD // NS
DUMMY = N


def _sc_scatter(x, src_p, dst_p, ea_p, zeros_big, zeros_se):
    mesh = plsc.VectorSubcoreMesh(core_axis_name="c", subcore_axis_name="s")

    @functools.partial(
        pl.kernel,
        out_type=(
            jax.ShapeDtypeStruct((NC, N_PAD, D), jnp.float32),
            jax.ShapeDtypeStruct((NC, N_PAD, DE), jnp.float32),
        ),
        mesh=mesh,
        scratch_types=[
            pltpu.VMEM((K,), jnp.int32),
            pltpu.VMEM((K,), jnp.int32),
            pltpu.VMEM((K, D), jnp.float32),
            pltpu.VMEM((K, DE), jnp.float32),
            pltpu.VMEM_SHARED((N_PAD, D), jnp.float32),
            pltpu.VMEM_SHARED((N_PAD, DE), jnp.float32),
            pltpu.SemaphoreType.DMA,
        ],
        compiler_params=pltpu.CompilerParams(use_tc_tiling_on_sc=False),
    )
    def sc_body(x_hbm, src_hbm, dst_hbm, ea_hbm, z_hbm, zse_hbm,
                agg_out, se_out, src_v, dst_v, rows_v, ea_v, agg_sh, se_sh, sem):
        cid = lax.axis_index("c")
        sid = lax.axis_index("s")
        wid = cid * NS + sid

        r0 = sid * ROWS_PER_TILE
        pltpu.sync_copy(z_hbm, rows_v)
        pltpu.sync_copy(zse_hbm, ea_v)
        for j, sz in ((0, K), (1, K), (2, K), (3, K), (4, ROWS_PER_TILE - 4 * K)):
            pltpu.sync_copy(rows_v.at[pl.ds(0, sz)],
                            agg_sh.at[pl.ds(r0 + j * K, sz)])
            pltpu.sync_copy(ea_v.at[pl.ds(0, sz)],
                            se_sh.at[pl.ds(r0 + j * K, sz)])
        plsc.subcore_barrier()

        def chunk(i, carry):
            off = (wid * CH_PER_TILE + i) * K
            pltpu.sync_copy(src_hbm.at[pl.ds(off, K)], src_v)
            pltpu.sync_copy(dst_hbm.at[pl.ds(off, K)], dst_v)
            pltpu.sync_copy(ea_hbm.at[pl.ds(off, K)], ea_v)
            pltpu.async_copy(x_hbm.at[src_v], rows_v, sem).wait()
            pltpu.sync_copy(rows_v, agg_sh.at[dst_v], add=True)
            pltpu.sync_copy(ea_v, se_sh.at[dst_v], add=True)
            return carry

        lax.fori_loop(0, CH_PER_TILE, chunk, 0)
        plsc.subcore_barrier()

        for j, sz in ((0, K), (1, K), (2, K), (3, K), (4, ROWS_PER_TILE - 4 * K)):
            pltpu.sync_copy(agg_sh.at[pl.ds(r0 + j * K, sz)],
                            rows_v.at[pl.ds(0, sz)])
            pltpu.sync_copy(rows_v.at[pl.ds(0, sz)],
                            agg_out.at[cid, pl.ds(r0 + j * K, sz)])
            pltpu.sync_copy(se_sh.at[pl.ds(r0 + j * K, sz)],
                            ea_v.at[pl.ds(0, sz)])
            pltpu.sync_copy(ea_v.at[pl.ds(0, sz)],
                            se_out.at[cid, pl.ds(r0 + j * K, sz)])

    return sc_body(x, src_p, dst_p, ea_p, zeros_big, zeros_se)


def _tc_body(aggp_ref, sep_ref, x_ref, We_ref, Wr_ref, br_ref, Wo_ref,
             g_ref, be_ref, out_ref):
    agg = aggp_ref[0, :N, :] + aggp_ref[1, :N, :]
    se = sep_ref[0, :N, :] + sep_ref[1, :N, :]
    x = x_ref[...]
    ea = lax.dot_general(se, We_ref[...], (((1,), (1,)), ((), ())),
                         preferred_element_type=jnp.float32)
    m = agg + ea
    pre = lax.dot_general(m, Wr_ref[...], (((1,), (1,)), ((), ())),
                          preferred_element_type=jnp.float32)
    pre = pre + lax.dot_general(x, Wo_ref[...], (((1,), (1,)), ((), ())),
                                preferred_element_type=jnp.float32)
    pre = pre + br_ref[...]
    pre = jnp.maximum(pre, 0.0)
    mean = jnp.mean(pre, axis=0, keepdims=True)
    var = jnp.mean((pre - mean) ** 2, axis=0, keepdims=True)
    out_ref[...] = (pre - mean) * lax.rsqrt(var + 1e-5) * g_ref[...] + be_ref[...]


def kernel(x, edge_index, edge_attr, W_edge, W_rel, b_rel, W_root, gamma, beta):
    src = edge_index[0].astype(jnp.int32)
    dst = edge_index[1].astype(jnp.int32)
    pad = E_PAD - E
    src_p = jnp.concatenate([src, jnp.zeros((pad,), jnp.int32)])
    dst_p = jnp.concatenate([dst, jnp.full((pad,), DUMMY, jnp.int32)])
    ea_p = jnp.concatenate([edge_attr, jnp.zeros((pad, DE), jnp.float32)])
    zeros_big = jnp.zeros((K, D), jnp.float32)
    zeros_se = jnp.zeros((K, DE), jnp.float32)

    aggp, sep = _sc_scatter(x, src_p, dst_p, ea_p, zeros_big, zeros_se)

    out = pl.pallas_call(
        _tc_body,
        out_shape=jax.ShapeDtypeStruct((N, D), jnp.float32),
    )(aggp, sep, x, W_edge, W_rel, b_rel.reshape(1, D), W_root,
      gamma.reshape(1, D), beta.reshape(1, D))
    return out
